# baseline (speedup 1.0000x reference)
import dataclasses
import functools

import jax
import jax.numpy as jnp
from jax import lax
from jax.experimental import pallas as pl
from jax.experimental.pallas import tpu as pltpu
from jax.experimental.pallas import tpu_sc as plsc

_SIGNIF_ARG = 0.1
_L = 16

_W1, _W2, _W3 = 11, 11, 9
_NB1, _NB2, _NB3 = 1 << _W1, 1 << _W2, 1 << _W3
_S1, _S2 = _W2 + _W3, _W3


def _sc_scan_topdown(hist_ref, nbins, rank, lane):
    def body(j, carry):
        found, b_sel, r_in, cum = carry
        base = nbins - (j + 1) * _L
        c = hist_ref[pl.ds(base, _L)]
        pre = plsc.cumsum(c)
        total = pre[_L - 1]
        suf = (total - pre) + c
        ge = (cum + suf) >= rank
        npc = plsc.all_reduce_population_count(ge)[0]
        hit = (found == 0) & (npc > 0)
        istar = npc - 1
        pre_i = jnp.sum(jnp.where(lane == istar, pre, 0))
        b_here = base + istar
        r_here = rank - (cum + (total - pre_i))
        return (
            jnp.where(hit, 1, found),
            jnp.where(hit, b_here, b_sel),
            jnp.where(hit, r_here, r_in),
            cum + total,
        )

    zero = jnp.int32(0)
    found, b_sel, r_in, _ = lax.fori_loop(
        0, nbins // _L, body, (zero, zero, zero, zero)
    )
    return found, b_sel, r_in


def _sc_select(hist_ref, totals_ref, nbins, rank, lane):
    nchunks = nbins // _L

    @plsc.parallel_loop(0, nchunks, unroll=8)
    def _(j):
        c = hist_ref[pl.ds(j * _L, _L)]
        totals_ref[j] = jnp.sum(c)

    def body(j, carry):
        found, jc, r_c, cum = carry
        jj = nchunks - 1 - j
        t = totals_ref[jj]
        hit = (found == 0) & ((cum + t) >= rank)
        return (
            jnp.where(hit, 1, found),
            jnp.where(hit, jj, jc),
            jnp.where(hit, rank - cum, r_c),
            cum + t,
        )

    zero = jnp.int32(0)
    found, jc, r_c, _ = lax.fori_loop(
        0, nchunks, body, (zero, zero, zero, zero), unroll=4
    )

    c = hist_ref[pl.ds(jc * _L, _L)]
    pre = plsc.cumsum(c)
    total = pre[_L - 1]
    suf = (total - pre) + c
    ge = suf >= r_c
    npc = plsc.all_reduce_population_count(ge)[0]
    istar = npc - 1
    pre_i = jnp.sum(jnp.where(lane == istar, pre, 0))
    b = jc * _L + istar
    r_in = r_c - (total - pre_i)
    return found, b, r_in


def _sc_zero(hist_ref, nbins):
    zeros = jnp.zeros((_L,), jnp.int32)

    @plsc.parallel_loop(0, nbins // _L, unroll=8)
    def _(i):
        hist_ref[pl.ds(i * _L, _L)] = zeros


def _sc_row_select(k, nsteps, row_f, hist, totals, lane, ones):
    _sc_zero(hist, _NB1)

    @plsc.parallel_loop(0, nsteps, unroll=16)
    def _(i):
        v = row_f[pl.ds(i * _L, _L)]
        b = plsc.bitcast(v, jnp.int32) & jnp.int32(0x7FFFFFFF)
        finite = (b & jnp.int32(0x7F800000)) != jnp.int32(0x7F800000)
        b = jnp.where(finite, b, jnp.int32(-1))
        row_f[pl.ds(i * _L, _L)] = plsc.bitcast(b, jnp.float32)
        plsc.addupdate_scatter(
            hist, [lax.shift_right_arithmetic(b, _S1)], ones, mask=finite
        )

    f1, b1, r1 = _sc_select(hist, totals, _NB1, jnp.int32(k), lane)

    _sc_zero(hist, _NB2)

    @plsc.parallel_loop(0, nsteps, unroll=16)
    def _(i):
        b = plsc.bitcast(row_f[pl.ds(i * _L, _L)], jnp.int32)
        m = lax.shift_right_arithmetic(b, _S1) == b1
        idx = lax.shift_right_arithmetic(b, _S2) & jnp.int32(_NB2 - 1)
        plsc.addupdate_scatter(hist, [idx], ones, mask=m)

    f2, b2, r2 = _sc_select(hist, totals, _NB2, r1, lane)

    _sc_zero(hist, _NB3)
    pfx2 = (b1 << _W2) | b2

    @plsc.parallel_loop(0, nsteps, unroll=16)
    def _(i):
        b = plsc.bitcast(row_f[pl.ds(i * _L, _L)], jnp.int32)
        m = lax.shift_right_arithmetic(b, _S2) == pfx2
        idx = b & jnp.int32(_NB3 - 1)
        plsc.addupdate_scatter(hist, [idx], ones, mask=m)

    f3, b3, _ = _sc_select(hist, totals, _NB3, r2, lane)

    thr = (pfx2 << _W3) | b3
    return jnp.where((f1 == 1) & (f2 == 1) & (f3 == 1), thr, 0)


def _sc_threshold_kernel(
    n, k, delta_hbm, thr_hbm, buf0, buf1, hist, totals, obuf, sem0, sem1
):
    nsteps = n // _L
    ones = jnp.full((_L,), 1, jnp.int32)
    lane = lax.iota(jnp.int32, _L)
    wid = lax.axis_index("subcore") * 2 + lax.axis_index("core")

    row0 = wid * 2
    row1 = row0 + 1
    copy0 = pltpu.async_copy(delta_hbm.at[row0], buf0, sem0)
    copy1 = pltpu.async_copy(delta_hbm.at[row1], buf1, sem1)

    copy0.wait()
    thr0 = _sc_row_select(k, nsteps, buf0, hist, totals, lane, ones)
    obuf[...] = ones * thr0
    out0 = pltpu.async_copy(obuf, thr_hbm.at[row0], sem0)
    out0.wait()

    copy1.wait()
    thr1 = _sc_row_select(k, nsteps, buf1, hist, totals, lane, ones)
    obuf[...] = ones * thr1
    pltpu.async_copy(obuf, thr_hbm.at[row1], sem1).wait()


def _aux_kernel(delta_ref, dir_ref, rank_ref, valid_ref):
    d = delta_ref[...]
    dir_ref[...] = (d > 0).astype(jnp.int32)
    rank_ref[...] = jnp.abs(d)
    valid_ref[...] = jnp.isfinite(d).astype(jnp.int8)


def _sig_kernel(delta_ref, thr_ref, sig_ref):
    d = delta_ref[...]
    thr = thr_ref[...][:, :1]
    valid = jnp.isfinite(d)
    bits = lax.bitcast_convert_type(jnp.abs(d), jnp.int32)
    bits = jnp.where(valid, bits, -1)
    sig_ref[...] = ((bits >= thr) & valid).astype(jnp.int8)


def kernel(delta):
    B, N = delta.shape
    k = max(1, int(N * _SIGNIF_ARG))

    mesh = plsc.VectorSubcoreMesh(
        core_axis_name="core", subcore_axis_name="subcore"
    )
    cp = pltpu.CompilerParams()
    if "needs_layout_passes" in pltpu.CompilerParams.__dataclass_fields__:
        cp = dataclasses.replace(cp, needs_layout_passes=False)
    sc_thresh = pl.kernel(
        functools.partial(_sc_threshold_kernel, N, k),
        out_type=jax.ShapeDtypeStruct((B, _L), jnp.int32),
        mesh=mesh,
        compiler_params=cp,
        scratch_types=[
            pltpu.VMEM((N,), jnp.float32),
            pltpu.VMEM((N,), jnp.float32),
            pltpu.VMEM((_NB1,), jnp.int32),
            pltpu.SMEM((_NB1 // _L,), jnp.int32),
            pltpu.VMEM((_L,), jnp.int32),
            pltpu.SemaphoreType.DMA,
            pltpu.SemaphoreType.DMA,
        ],
    )
    thr = sc_thresh(delta)

    blk = 8192
    de_dir, rank_score, valid_mask = pl.pallas_call(
        _aux_kernel,
        grid=(N // blk,),
        in_specs=[pl.BlockSpec((B, blk), lambda j: (0, j))],
        out_specs=[pl.BlockSpec((B, blk), lambda j: (0, j)) for _ in range(3)],
        out_shape=[
            jax.ShapeDtypeStruct((B, N), jnp.int32),
            jax.ShapeDtypeStruct((B, N), jnp.float32),
            jax.ShapeDtypeStruct((B, N), jnp.int8),
        ],
    )(delta)

    de_sig = pl.pallas_call(
        _sig_kernel,
        grid=(N // blk,),
        in_specs=[
            pl.BlockSpec((B, blk), lambda j: (0, j)),
            pl.BlockSpec((B, _L), lambda j: (0, 0)),
        ],
        out_specs=pl.BlockSpec((B, blk), lambda j: (0, j)),
        out_shape=jax.ShapeDtypeStruct((B, N), jnp.int8),
    )(delta, thr)
    return (
        de_sig.astype(jnp.bool_),
        de_dir.astype(jnp.int64),
        rank_score,
        valid_mask.astype(jnp.bool_),
    )


# speedup vs baseline: 1.3552x; 1.3552x over previous
import dataclasses
import functools

import jax
import jax.numpy as jnp
from jax import lax
from jax.experimental import pallas as pl
from jax.experimental.pallas import tpu as pltpu
from jax.experimental.pallas import tpu_sc as plsc

_SIGNIF_ARG = 0.1
_L = 16

_W1, _W2, _W3 = 11, 11, 9
_NB1, _NB2, _NB3 = 1 << _W1, 1 << _W2, 1 << _W3
_S1, _S2 = _W2 + _W3, _W3


def _sc_scan_topdown(hist_ref, nbins, rank, lane):
    def body(j, carry):
        found, b_sel, r_in, cum = carry
        base = nbins - (j + 1) * _L
        c = hist_ref[pl.ds(base, _L)]
        pre = plsc.cumsum(c)
        total = pre[_L - 1]
        suf = (total - pre) + c
        ge = (cum + suf) >= rank
        npc = plsc.all_reduce_population_count(ge)[0]
        hit = (found == 0) & (npc > 0)
        istar = npc - 1
        pre_i = jnp.sum(jnp.where(lane == istar, pre, 0))
        b_here = base + istar
        r_here = rank - (cum + (total - pre_i))
        return (
            jnp.where(hit, 1, found),
            jnp.where(hit, b_here, b_sel),
            jnp.where(hit, r_here, r_in),
            cum + total,
        )

    zero = jnp.int32(0)
    found, b_sel, r_in, _ = lax.fori_loop(
        0, nbins // _L, body, (zero, zero, zero, zero)
    )
    return found, b_sel, r_in


def _sc_select(hist_ref, totals_ref, nbins, rank, lane):
    nchunks = nbins // _L

    @plsc.parallel_loop(0, nchunks, unroll=8)
    def _(j):
        c = hist_ref[pl.ds(j * _L, _L)]
        totals_ref[j] = jnp.sum(c)

    def body(j, carry):
        found, jc, r_c, cum = carry
        jj = nchunks - 1 - j
        t = totals_ref[jj]
        hit = (found == 0) & ((cum + t) >= rank)
        return (
            jnp.where(hit, 1, found),
            jnp.where(hit, jj, jc),
            jnp.where(hit, rank - cum, r_c),
            cum + t,
        )

    zero = jnp.int32(0)
    found, jc, r_c, _ = lax.fori_loop(
        0, nchunks, body, (zero, zero, zero, zero), unroll=4
    )

    c = hist_ref[pl.ds(jc * _L, _L)]
    pre = plsc.cumsum(c)
    total = pre[_L - 1]
    suf = (total - pre) + c
    ge = suf >= r_c
    npc = plsc.all_reduce_population_count(ge)[0]
    istar = npc - 1
    pre_i = jnp.sum(jnp.where(lane == istar, pre, 0))
    b = jc * _L + istar
    r_in = r_c - (total - pre_i)
    return found, b, r_in


def _sc_zero(hist_ref, nbins):
    zeros = jnp.zeros((_L,), jnp.int32)

    @plsc.parallel_loop(0, nbins // _L, unroll=8)
    def _(i):
        hist_ref[pl.ds(i * _L, _L)] = zeros


def _sc_row_select(k, nsteps, row_f, bits_v, hist, totals, lane, ones):
    _sc_zero(hist, _NB1)

    @plsc.parallel_loop(0, nsteps, unroll=8)
    def _(i):
        v = row_f[pl.ds(i * _L, _L)]
        b = plsc.bitcast(v, jnp.int32) & jnp.int32(0x7FFFFFFF)
        finite = (b & jnp.int32(0x7F800000)) != jnp.int32(0x7F800000)
        b = jnp.where(finite, b, jnp.int32(-1))
        bits_v[pl.ds(i * _L, _L)] = b
        plsc.addupdate_scatter(
            hist, [lax.shift_right_arithmetic(b, _S1)], ones, mask=finite
        )

    f1, b1, r1 = _sc_select(hist, totals, _NB1, jnp.int32(k), lane)

    _sc_zero(hist, _NB2)

    @plsc.parallel_loop(0, nsteps, unroll=8)
    def _(i):
        b = bits_v[pl.ds(i * _L, _L)]
        m = lax.shift_right_arithmetic(b, _S1) == b1
        idx = lax.shift_right_arithmetic(b, _S2) & jnp.int32(_NB2 - 1)
        plsc.addupdate_scatter(hist, [idx], ones, mask=m)

    f2, b2, r2 = _sc_select(hist, totals, _NB2, r1, lane)

    _sc_zero(hist, _NB3)
    pfx2 = (b1 << _W2) | b2

    @plsc.parallel_loop(0, nsteps, unroll=8)
    def _(i):
        b = bits_v[pl.ds(i * _L, _L)]
        m = lax.shift_right_arithmetic(b, _S2) == pfx2
        idx = b & jnp.int32(_NB3 - 1)
        plsc.addupdate_scatter(hist, [idx], ones, mask=m)

    f3, b3, _ = _sc_select(hist, totals, _NB3, r2, lane)

    thr = (pfx2 << _W3) | b3
    return jnp.where((f1 == 1) & (f2 == 1) & (f3 == 1), thr, 0)


def _sc_threshold_kernel(
    n, k, delta_hbm, thr_hbm, row_f, bits_v, hist, totals, obuf, sem
):
    nsteps = n // _L
    ones = jnp.full((_L,), 1, jnp.int32)
    lane = lax.iota(jnp.int32, _L)
    wid = lax.axis_index("subcore") * 2 + lax.axis_index("core")

    for rlocal in range(2):
        row = wid * 2 + rlocal
        pltpu.async_copy(delta_hbm.at[row], row_f, sem).wait()
        thr = _sc_row_select(k, nsteps, row_f, bits_v, hist, totals, lane, ones)
        obuf[...] = ones * thr
        pltpu.async_copy(obuf, thr_hbm.at[row], sem).wait()


def _aux_kernel(delta_ref, dir_ref, rank_ref, valid_ref):
    d = delta_ref[...]
    dir_ref[...] = (d > 0).astype(jnp.int32)
    rank_ref[...] = jnp.abs(d)
    valid_ref[...] = jnp.isfinite(d).astype(jnp.int8)


def _sig_kernel(delta_ref, thr_ref, sig_ref):
    d = delta_ref[...]
    thr = thr_ref[...][:, :1]
    valid = jnp.isfinite(d)
    bits = lax.bitcast_convert_type(jnp.abs(d), jnp.int32)
    bits = jnp.where(valid, bits, -1)
    sig_ref[...] = ((bits >= thr) & valid).astype(jnp.int8)


def kernel(delta):
    B, N = delta.shape
    k = max(1, int(N * _SIGNIF_ARG))

    mesh = plsc.VectorSubcoreMesh(
        core_axis_name="core", subcore_axis_name="subcore"
    )
    cp = pltpu.CompilerParams()
    if "needs_layout_passes" in pltpu.CompilerParams.__dataclass_fields__:
        cp = dataclasses.replace(cp, needs_layout_passes=False)
    sc_thresh = pl.kernel(
        functools.partial(_sc_threshold_kernel, N, k),
        out_type=jax.ShapeDtypeStruct((B, _L), jnp.int32),
        mesh=mesh,
        compiler_params=cp,
        scratch_types=[
            pltpu.VMEM((N,), jnp.float32),
            pltpu.VMEM((N,), jnp.int32),
            pltpu.VMEM((_NB1,), jnp.int32),
            pltpu.SMEM((_NB1 // _L,), jnp.int32),
            pltpu.VMEM((_L,), jnp.int32),
            pltpu.SemaphoreType.DMA,
        ],
    )
    thr = sc_thresh(delta)

    blk = 8192
    de_dir, rank_score, valid_mask = pl.pallas_call(
        _aux_kernel,
        grid=(N // blk,),
        in_specs=[pl.BlockSpec((B, blk), lambda j: (0, j))],
        out_specs=[pl.BlockSpec((B, blk), lambda j: (0, j)) for _ in range(3)],
        out_shape=[
            jax.ShapeDtypeStruct((B, N), jnp.int32),
            jax.ShapeDtypeStruct((B, N), jnp.float32),
            jax.ShapeDtypeStruct((B, N), jnp.int8),
        ],
    )(delta)

    de_sig = pl.pallas_call(
        _sig_kernel,
        grid=(N // blk,),
        in_specs=[
            pl.BlockSpec((B, blk), lambda j: (0, j)),
            pl.BlockSpec((B, _L), lambda j: (0, 0)),
        ],
        out_specs=pl.BlockSpec((B, blk), lambda j: (0, j)),
        out_shape=jax.ShapeDtypeStruct((B, N), jnp.int8),
    )(delta, thr)
    return (
        de_sig.astype(jnp.bool_),
        de_dir.astype(jnp.int64),
        rank_score,
        valid_mask.astype(jnp.bool_),
    )
